# baseline (speedup 1.0000x reference)
import functools

import jax
import jax.numpy as jnp
from jax.experimental import pallas as pl
from jax.experimental.pallas import tpu as pltpu
from jax.experimental.pallas import tpu_sc as plsc


def _dist_body(z_ref, cb_ref, idx_ref, loss_ref):
    b = pl.program_id(0)
    zb = z_ref[0]
    cb = cb_ref[...]

    cnorm = jnp.sum(cb * cb, axis=1)
    scores = jax.lax.dot_general(
        cb, zb, (((1,), (0,)), ((), ())),
        preferred_element_type=jnp.float32)
    d = cnorm[:, None] - 2.0 * scores

    dmin = jnp.min(d, axis=0)
    idx = jnp.argmin(d, axis=0).astype(jnp.int32)

    xnorm = jnp.sum(zb * zb, axis=0)
    loss_part = jnp.sum(dmin + xnorm)

    idx_ref[0, 0] = idx
    loss_blk = jnp.reshape(loss_part, (1, 1))

    @pl.when(b == 0)
    def _init():
        loss_ref[...] = loss_blk

    @pl.when(b > 0)
    def _acc():
        loss_ref[...] += loss_blk


def _dist_argmin(z3, codebook):
    B, C, N = z3.shape
    K = codebook.shape[0]
    return pl.pallas_call(
        _dist_body,
        grid=(B,),
        in_specs=[
            pl.BlockSpec((1, C, N), lambda b: (b, 0, 0)),
            pl.BlockSpec((K, C), lambda b: (0, 0)),
        ],
        out_specs=[
            pl.BlockSpec((1, 1, N), lambda b: (b, 0, 0)),
            pl.BlockSpec((1, 1), lambda b: (0, 0)),
        ],
        out_shape=[
            jax.ShapeDtypeStruct((B, 1, N), jnp.int32),
            jax.ShapeDtypeStruct((1, 1), jnp.float32),
        ],
    )(z3, codebook)


_SC_CORES = 2
_SC_SUBCORES = 16
_SC_WORKERS = _SC_CORES * _SC_SUBCORES


def _sc_gather4d(cbT, idx, H, W):
    C, K = cbT.shape
    B, N = idx.shape
    CPW = C // 2
    RCH = 16
    L = 16
    mesh = plsc.VectorSubcoreMesh(core_axis_name="c", subcore_axis_name="s")

    @functools.partial(
        pl.kernel, mesh=mesh,
        out_type=jax.ShapeDtypeStruct((B, C, H, W), jnp.float32),
        compiler_params=pltpu.CompilerParams(needs_layout_passes=False),
        scratch_types=[
            pltpu.VMEM((N,), jnp.int32),
            pltpu.VMEM((RCH, K), jnp.float32),
            pltpu.VMEM((RCH, H, W), jnp.float32),
        ],
    )
    def k(cbT_hbm, idx_hbm, out_hbm, idx_v, rows_v, out_v):
        wid = jax.lax.axis_index("s") * _SC_CORES + jax.lax.axis_index("c")
        b = wid // 2
        ci0 = (wid % 2) * CPW
        pltpu.sync_copy(idx_hbm.at[b], idx_v)
        for ch in range(CPW // RCH):
            pltpu.sync_copy(cbT_hbm.at[pl.ds(ci0 + ch * RCH, RCH)], rows_v)

            def body(j, carry):
                iv = idx_v[pl.ds(j * L, L)]
                hv = jnp.full((L,), j // (W // L), jnp.int32)
                wv = (jax.lax.iota(jnp.int32, L)
                      + (j % (W // L)) * L)
                for ci in range(RCH):
                    civ = jnp.full((L,), ci, jnp.int32)
                    vals = plsc.load_gather(rows_v, [civ, iv])
                    plsc.store_scatter(out_v, [civ, hv, wv], vals)
                return carry

            jax.lax.fori_loop(0, N // L, body, 0)
            pltpu.sync_copy(out_v, out_hbm.at[b, pl.ds(ci0 + ch * RCH, RCH)])

    return k(cbT, idx)


@jax.jit
def _vq(z, codebook):
    b, c, h, w = z.shape
    z3 = z.reshape(b, c, h * w)
    idx, loss = _dist_argmin(z3, codebook)
    z_q_out = _sc_gather4d(codebook.T, idx.reshape(b, h * w), h, w)
    codebook_loss = loss[0, 0] * 1.25 / (b * c * h * w)
    indices_out = idx.reshape(b, 1, h, w)
    return (z_q_out, codebook_loss, indices_out)


def kernel(z, embedding_weight):
    return _vq(z, embedding_weight)


# speedup vs baseline: 1.9924x; 1.9924x over previous
import functools

import jax
import jax.numpy as jnp
from jax.experimental import pallas as pl
from jax.experimental.pallas import tpu as pltpu
from jax.experimental.pallas import tpu_sc as plsc


def _dist_body(z_ref, cb_ref, idx_ref, loss_ref):
    b = pl.program_id(0)
    zb = z_ref[0]
    cb = cb_ref[...]

    cnorm = jnp.sum(cb * cb, axis=1)
    scores = jax.lax.dot_general(
        cb, zb, (((1,), (0,)), ((), ())),
        preferred_element_type=jnp.float32)
    d = cnorm[:, None] - 2.0 * scores

    dmin = jnp.min(d, axis=0)
    idx = jnp.argmin(d, axis=0).astype(jnp.int32)

    xnorm = jnp.sum(zb * zb, axis=0)
    loss_part = jnp.sum(dmin + xnorm)

    idx_ref[0, 0] = idx
    loss_blk = jnp.reshape(loss_part, (1, 1))

    @pl.when(b == 0)
    def _init():
        loss_ref[...] = loss_blk

    @pl.when(b > 0)
    def _acc():
        loss_ref[...] += loss_blk


def _dist_argmin(z3, codebook):
    B, C, N = z3.shape
    K = codebook.shape[0]
    return pl.pallas_call(
        _dist_body,
        grid=(B,),
        in_specs=[
            pl.BlockSpec((1, C, N), lambda b: (b, 0, 0)),
            pl.BlockSpec((K, C), lambda b: (0, 0)),
        ],
        out_specs=[
            pl.BlockSpec((1, 1, N), lambda b: (b, 0, 0)),
            pl.BlockSpec((1, 1), lambda b: (0, 0)),
        ],
        out_shape=[
            jax.ShapeDtypeStruct((B, 1, N), jnp.int32),
            jax.ShapeDtypeStruct((1, 1), jnp.float32),
        ],
    )(z3, codebook)


def _onehot_body(idx_ref, cb_ref, zq_ref):
    idx = idx_ref[0, 0]
    cb = cb_ref[...]
    K = cb.shape[0]
    N = idx.shape[0]
    iota_k = jax.lax.broadcasted_iota(jnp.int32, (K, N), 0)
    onehot = (iota_k == idx[None, :]).astype(jnp.bfloat16)
    zq_ref[0] = jax.lax.dot_general(
        cb.astype(jnp.bfloat16), onehot, (((0,), (0,)), ((), ())),
        preferred_element_type=jnp.float32)


def _onehot_zq(idx, codebook, N):
    B = idx.shape[0]
    K, C = codebook.shape
    return pl.pallas_call(
        _onehot_body,
        grid=(B,),
        in_specs=[
            pl.BlockSpec((1, 1, N), lambda b: (b, 0, 0)),
            pl.BlockSpec((K, C), lambda b: (0, 0)),
        ],
        out_specs=pl.BlockSpec((1, C, N), lambda b: (b, 0, 0)),
        out_shape=jax.ShapeDtypeStruct((B, C, N), jnp.float32),
    )(idx, codebook)


_SC_CORES = 2
_SC_SUBCORES = 16
_SC_WORKERS = _SC_CORES * _SC_SUBCORES


def _sc_gather(codebook, idx_flat):
    T = idx_flat.shape[0]
    C = codebook.shape[1]
    per_w = T // _SC_WORKERS
    mesh = plsc.VectorSubcoreMesh(core_axis_name="c", subcore_axis_name="s")

    @functools.partial(
        pl.kernel, mesh=mesh,
        out_type=jax.ShapeDtypeStruct((T, C), jnp.float32),
        scratch_types=[
            pltpu.VMEM((per_w,), jnp.int32),
            pltpu.VMEM((per_w, C), jnp.float32),
            pltpu.SemaphoreType.DMA,
        ],
    )
    def k(table_hbm, idx_hbm, out_hbm, idx_v, rows_v, sem):
        wid = jax.lax.axis_index("s") * _SC_CORES + jax.lax.axis_index("c")
        base = wid * per_w
        pltpu.sync_copy(idx_hbm.at[pl.ds(base, per_w)], idx_v)
        pltpu.async_copy(table_hbm.at[idx_v], rows_v, sem).wait()
        pltpu.sync_copy(rows_v, out_hbm.at[pl.ds(base, per_w)])

    return k(codebook, idx_flat)


@jax.jit
def _vq(z, codebook):
    b, c, h, w = z.shape
    z3 = z.reshape(b, c, h * w)
    idx, loss = _dist_argmin(z3, codebook)
    b_lo = b // 2
    zq_lo = _onehot_zq(idx[:b_lo], codebook, h * w).reshape(b_lo, c, h, w)
    zq_hi_flat = _sc_gather(codebook, idx[b_lo:].reshape(-1))
    zq_hi = zq_hi_flat.reshape(b - b_lo, h, w, c).transpose(0, 3, 1, 2)
    z_q_out = jnp.concatenate([zq_lo, zq_hi], axis=0)
    codebook_loss = loss[0, 0] * 1.25 / (b * c * h * w)
    indices_out = idx.reshape(b, 1, h, w)
    return (z_q_out, codebook_loss, indices_out)


def kernel(z, embedding_weight):
    return _vq(z, embedding_weight)


# speedup vs baseline: 2.4323x; 1.2208x over previous
import functools

import jax
import jax.numpy as jnp
from jax.experimental import pallas as pl
from jax.experimental.pallas import tpu as pltpu
from jax.experimental.pallas import tpu_sc as plsc


def _dist_body(z_ref, cb_ref, idx_ref, loss_ref):
    b = pl.program_id(0)
    zb = z_ref[0]
    cb = cb_ref[...]
    N = zb.shape[1]
    NT = N // 2

    cnorm = jnp.sum(cb * cb, axis=1)
    loss_part = jnp.float32(0.0)
    for t in range(2):
        zt = zb[:, t * NT:(t + 1) * NT]
        scores = jax.lax.dot_general(
            cb, zt, (((1,), (0,)), ((), ())),
            preferred_element_type=jnp.float32)
        d = cnorm[:, None] - 2.0 * scores
        dmin = jnp.min(d, axis=0)
        idx = jnp.argmin(d, axis=0).astype(jnp.int32)
        xnorm = jnp.sum(zt * zt, axis=0)
        loss_part = loss_part + jnp.sum(dmin + xnorm)
        idx_ref[0, 0, pl.ds(t * NT, NT)] = idx

    loss_blk = jnp.reshape(loss_part, (1, 1))

    @pl.when(b == 0)
    def _init():
        loss_ref[...] = loss_blk

    @pl.when(b > 0)
    def _acc():
        loss_ref[...] += loss_blk


def _dist_argmin(z3, codebook):
    B, C, N = z3.shape
    K = codebook.shape[0]
    return pl.pallas_call(
        _dist_body,
        grid=(B,),
        in_specs=[
            pl.BlockSpec((1, C, N), lambda b: (b, 0, 0)),
            pl.BlockSpec((K, C), lambda b: (0, 0)),
        ],
        out_specs=[
            pl.BlockSpec((1, 1, N), lambda b: (b, 0, 0)),
            pl.BlockSpec((1, 1), lambda b: (0, 0)),
        ],
        out_shape=[
            jax.ShapeDtypeStruct((B, 1, N), jnp.int32),
            jax.ShapeDtypeStruct((1, 1), jnp.float32),
        ],
    )(z3, codebook)


_SC_CORES = 2
_SC_SUBCORES = 16
_SC_WORKERS = _SC_CORES * _SC_SUBCORES


def _sc_gather(codebook, idx_flat):
    T = idx_flat.shape[0]
    C = codebook.shape[1]
    per_w = T // _SC_WORKERS
    CH = 128
    n_ch = per_w // CH
    mesh = plsc.VectorSubcoreMesh(core_axis_name="c", subcore_axis_name="s")

    @functools.partial(
        pl.kernel, mesh=mesh,
        out_type=jax.ShapeDtypeStruct((T, C), jnp.float32),
        scratch_types=[
            pltpu.VMEM((per_w,), jnp.int32),
            pltpu.VMEM((2, CH, C), jnp.float32),
            pltpu.SemaphoreType.DMA((2,)),
            pltpu.SemaphoreType.DMA((2,)),
        ],
    )
    def k(table_hbm, idx_hbm, out_hbm, idx_v, rows_v, gsem, osem):
        wid = jax.lax.axis_index("s") * _SC_CORES + jax.lax.axis_index("c")
        base = wid * per_w
        pltpu.sync_copy(idx_hbm.at[pl.ds(base, per_w)], idx_v)

        def gather(ch, buf):
            return pltpu.make_async_copy(
                table_hbm.at[idx_v.at[pl.ds(ch * CH, CH)]],
                rows_v.at[buf], gsem.at[buf])

        def put(ch, buf):
            return pltpu.make_async_copy(
                rows_v.at[buf], out_hbm.at[pl.ds(base + ch * CH, CH)],
                osem.at[buf])

        gather(0, 0).start()
        for ch in range(n_ch):
            cur = ch % 2
            nxt = 1 - cur
            gather(ch, cur).wait()
            if ch + 1 < n_ch:
                if ch >= 1:
                    put(ch - 1, nxt).wait()
                gather(ch + 1, nxt).start()
            put(ch, cur).start()
        for ch in (n_ch - 2, n_ch - 1):
            put(ch, ch % 2).wait()

    return k(codebook, idx_flat)


@jax.jit
def _vq(z, codebook):
    b, c, h, w = z.shape
    z3 = z.reshape(b, c, h * w)
    idx, loss = _dist_argmin(z3, codebook)
    zq_flat = _sc_gather(codebook, idx.reshape(-1))
    z_q_out = zq_flat.reshape(b, h, w, c).transpose(0, 3, 1, 2)
    codebook_loss = loss[0, 0] * 1.25 / (b * c * h * w)
    indices_out = idx.reshape(b, 1, h, w)
    return (z_q_out, codebook_loss, indices_out)


def kernel(z, embedding_weight):
    return _vq(z, embedding_weight)
